# baseline (speedup 1.0000x reference)
import functools

import jax
import jax.numpy as jnp
from jax.experimental import pallas as pl
from jax.experimental.pallas import tpu as pltpu

K = 64
DEPTH = 8
LANES = 128
R = 8


_SORT8 = [(0, 1), (2, 3), (4, 5), (6, 7),
          (0, 2), (1, 3), (4, 6), (5, 7),
          (1, 2), (5, 6),
          (0, 4), (1, 5), (2, 6), (3, 7),
          (2, 4), (3, 5),
          (1, 2), (3, 4), (5, 6)]
_BITONIC8 = [(0, 4), (1, 5), (2, 6), (3, 7),
             (0, 2), (1, 3), (4, 6), (5, 7),
             (0, 1), (2, 3), (4, 5), (6, 7)]


def _cswap(v, iv, i, j):
    ge = v[i] >= v[j]
    vi = jnp.where(ge, v[i], v[j])
    vj = jnp.where(ge, v[j], v[i])
    ii = jnp.where(ge, iv[i], iv[j])
    ij = jnp.where(ge, iv[j], iv[i])
    v[i], v[j], iv[i], iv[j] = vi, vj, ii, ij


def _topk_subtile(r, s_ref, vals_ref, idx_ref, cols):
    rows = pl.ds(r * R, R)
    lane = jax.lax.broadcasted_iota(jnp.int32, (R, LANES), 1)

    def insert_group(g, carry):
        vs, ids = carry
        slab = s_ref[rows, pl.ds(g * DEPTH * LANES, DEPTH * LANES)]
        xs = [slab[:, t * LANES:(t + 1) * LANES] for t in range(DEPTH)]
        xis = [jnp.full((R, LANES), g * DEPTH + t, jnp.int32)
               for t in range(DEPTH)]
        for i, j in _SORT8:
            _cswap(xs, xis, i, j)
        mv, mi = [], []
        for d in range(DEPTH):
            ge = vs[d] >= xs[DEPTH - 1 - d]
            mv.append(jnp.where(ge, vs[d], xs[DEPTH - 1 - d]))
            mi.append(jnp.where(ge, ids[d], xis[DEPTH - 1 - d]))
        for i, j in _BITONIC8:
            _cswap(mv, mi, i, j)
        return tuple(mv), tuple(mi)

    neg = jnp.full((R, LANES), -1.0, jnp.float32)
    zero_i = jnp.zeros((R, LANES), jnp.int32)
    vs, ids = jax.lax.fori_loop(
        0, cols // DEPTH, insert_group,
        (tuple(neg for _ in range(DEPTH)), tuple(zero_i for _ in range(DEPTH))))

    col = lane

    def extract(t, carry):
        h, ha, dc, acc_v, acc_i = carry
        g = h
        for sh in (1, 2, 4, 8, 16, 32, 64):
            g = jnp.maximum(g, pltpu.roll(g, sh, 1))
        f = ha * LANES + lane
        big = jnp.int32(2 ** 30)
        eq = h == g
        flat = jnp.where(eq, f, big)
        for sh in (1, 2, 4, 8, 16, 32, 64):
            flat = jnp.minimum(flat, pltpu.roll(flat, sh, 1))
        sel = eq & (f == flat)
        acc_v = jnp.where(col == t, g, acc_v)
        acc_i = jnp.where(col == t, flat, acc_i)
        nh = jnp.full((R, LANES), -1.0, jnp.float32)
        nhi = zero_i
        for d in range(DEPTH - 1, 0, -1):
            is_d = dc == d
            nh = jnp.where(is_d, vs[d], nh)
            nhi = jnp.where(is_d, ids[d], nhi)
        h = jnp.where(sel, nh, h)
        ha = jnp.where(sel, nhi, ha)
        dc = dc + sel.astype(jnp.int32)
        return h, ha, dc, acc_v, acc_i

    acc_v0 = jnp.zeros((R, LANES), jnp.float32)
    acc_i0 = jnp.zeros((R, LANES), jnp.int32)
    dc0 = jnp.ones((R, LANES), jnp.int32)
    _, _, _, acc_v, acc_i = jax.lax.fori_loop(
        0, K // 4, lambda t4, c: extract(
            4 * t4 + 3, extract(4 * t4 + 2, extract(
                4 * t4 + 1, extract(4 * t4, c)))),
        (vs[0], ids[0], dc0, acc_v0, acc_i0))
    vals_ref[rows, :] = acc_v[:, :K]
    idx_ref[rows, :] = acc_i[:, :K]
    return r + 1


def _body(a_ref, wt_ref, be_ref, dt_ref, bd_ref,
          recon_ref, vals_ref, idx_ref, s_ref, *, ne, mblk, b):
    j = pl.program_id(1)

    @pl.when(j < ne)
    def _encode():
        jj = jnp.minimum(j, ne - 1)
        s = jnp.dot(a_ref[...], wt_ref[...], preferred_element_type=jnp.float32)
        s = s + be_ref[pl.ds(jj * mblk, mblk)][None, :]
        s_ref[:, pl.ds(jj * mblk, mblk)] = jnp.maximum(s, 0.0)

    @pl.when(j == ne)
    def _topk():
        cols = (ne * mblk) // LANES
        jax.lax.fori_loop(
            0, b // R,
            lambda r, c: _topk_subtile(r, s_ref, vals_ref, idx_ref, cols),
            0)

    @pl.when(j >= ne)
    def _decode():
        jj = jnp.maximum(j - ne, 0)
        v64 = vals_ref[:, pl.ds(K - 1, 1)]
        sb = s_ref[:, pl.ds(jj * mblk, mblk)]
        sm = jnp.where(sb >= v64, sb, 0.0).astype(jnp.bfloat16)
        acc = jnp.dot(sm, dt_ref[...], preferred_element_type=jnp.float32)

        @pl.when(j == ne)
        def _():
            recon_ref[...] = bd_ref[...][None, :] + acc

        @pl.when(j > ne)
        def _():
            recon_ref[...] += acc


def _fused_call(a16, wt, b_e, dt, b_d, *, n, c, m, b, mblk, interpret=False):
    ne = m // mblk
    grid = (n // b, 2 * ne)
    body = functools.partial(_body, ne=ne, mblk=mblk, b=b)
    return pl.pallas_call(
        body,
        grid=grid,
        in_specs=[
            pl.BlockSpec((b, c), lambda i, j: (i, 0)),
            pl.BlockSpec((c, mblk), lambda i, j, ne=ne: (0, jnp.minimum(j, ne - 1))),
            pl.BlockSpec((m,), lambda i, j: (0,)),
            pl.BlockSpec((mblk, c), lambda i, j, ne=ne: (jnp.maximum(j - ne, 0), 0)),
            pl.BlockSpec((c,), lambda i, j: (0,)),
        ],
        out_specs=[
            pl.BlockSpec((b, c), lambda i, j: (i, 0)),
            pl.BlockSpec((b, K), lambda i, j: (i, 0)),
            pl.BlockSpec((b, K), lambda i, j: (i, 0)),
        ],
        out_shape=[
            jax.ShapeDtypeStruct((n, c), jnp.float32),
            jax.ShapeDtypeStruct((n, K), jnp.float32),
            jax.ShapeDtypeStruct((n, K), jnp.int32),
        ],
        scratch_shapes=[pltpu.VMEM((b, m), jnp.float32)],
        compiler_params=pltpu.CompilerParams(
            dimension_semantics=("parallel", "arbitrary")),
        interpret=interpret,
    )(a16, wt, b_e, dt, b_d)


def kernel(a, W_e, b_e, D, b_d, *, interpret=False, b=256, mblk=1024):
    n, c = a.shape
    m = W_e.shape[0]
    a16 = a.astype(jnp.bfloat16)
    wt = W_e.T.astype(jnp.bfloat16)
    dt = D.T.astype(jnp.bfloat16)
    recon, vals, idx = _fused_call(a16, wt, b_e, dt, b_d,
                                   n=n, c=c, m=m, b=b, mblk=mblk,
                                   interpret=interpret)
    return (recon, vals, idx)


# speedup vs baseline: 10.0646x; 10.0646x over previous
import functools

import jax
import jax.numpy as jnp
from jax.experimental import pallas as pl
from jax.experimental.pallas import tpu as pltpu

K = 64
DEPTH = 8
LANES = 128
R = 8


_SORT8 = [(0, 1), (2, 3), (4, 5), (6, 7),
          (0, 2), (1, 3), (4, 6), (5, 7),
          (1, 2), (5, 6),
          (0, 4), (1, 5), (2, 6), (3, 7),
          (2, 4), (3, 5),
          (1, 2), (3, 4), (5, 6)]
_BITONIC8 = [(0, 4), (1, 5), (2, 6), (3, 7),
             (0, 2), (1, 3), (4, 6), (5, 7),
             (0, 1), (2, 3), (4, 5), (6, 7)]


def _cswap(v, iv, i, j):
    ge = v[i] >= v[j]
    vi = jnp.where(ge, v[i], v[j])
    vj = jnp.where(ge, v[j], v[i])
    ii = jnp.where(ge, iv[i], iv[j])
    ij = jnp.where(ge, iv[j], iv[i])
    v[i], v[j], iv[i], iv[j] = vi, vj, ii, ij


RQ = 4
RW = R * RQ


def _insert_lists(rows, s_ref, cols):
    def insert_group(g, carry):
        vs, ids = carry
        slab = s_ref[rows, pl.ds(g * DEPTH * LANES, DEPTH * LANES)]
        xs = [slab[:, t * LANES:(t + 1) * LANES] for t in range(DEPTH)]
        xis = [jnp.full((R, LANES), g * DEPTH + t, jnp.int32)
               for t in range(DEPTH)]
        for i, j in _SORT8:
            _cswap(xs, xis, i, j)
        mv, mi = [], []
        for d in range(DEPTH):
            ge = vs[d] >= xs[DEPTH - 1 - d]
            mv.append(jnp.where(ge, vs[d], xs[DEPTH - 1 - d]))
            mi.append(jnp.where(ge, ids[d], xis[DEPTH - 1 - d]))
        for i, j in _BITONIC8:
            _cswap(mv, mi, i, j)
        return tuple(mv), tuple(mi)

    neg = jnp.full((R, LANES), -1.0, jnp.float32)
    zero_i = jnp.zeros((R, LANES), jnp.int32)
    return jax.lax.fori_loop(
        0, cols // DEPTH, insert_group,
        (tuple(neg for _ in range(DEPTH)), tuple(zero_i for _ in range(DEPTH))))


def _topk_subtile(r, s_ref, vals_ref, idx_ref, cols):
    qlists = [_insert_lists(pl.ds((r * RQ + q) * R, R), s_ref, cols)
              for q in range(RQ)]
    vs = tuple(jnp.concatenate([qlists[q][0][d] for q in range(RQ)], axis=0)
               for d in range(DEPTH))
    ids = tuple(jnp.concatenate([qlists[q][1][d] for q in range(RQ)], axis=0)
                for d in range(DEPTH))

    lane = jax.lax.broadcasted_iota(jnp.int32, (RW, LANES), 1)
    col64 = jax.lax.broadcasted_iota(jnp.int32, (RW, K), 1)
    zero_i = jnp.zeros((RW, LANES), jnp.int32)

    def extract(t, carry):
        h, ha, dc, acc_v, acc_i = carry
        g = jnp.max(h, axis=1, keepdims=True)
        f = ha * LANES + lane
        big = jnp.int32(2 ** 30)
        flat = jnp.min(jnp.where(h == g, f, big), axis=1, keepdims=True)
        sel = (h == g) & (f == flat)
        acc_v = jnp.where(col64 == t, g, acc_v)
        acc_i = jnp.where(col64 == t, flat, acc_i)
        nh = jnp.full((RW, LANES), -1.0, jnp.float32)
        nhi = zero_i
        for d in range(DEPTH - 1, 0, -1):
            is_d = dc == d
            nh = jnp.where(is_d, vs[d], nh)
            nhi = jnp.where(is_d, ids[d], nhi)
        h = jnp.where(sel, nh, h)
        ha = jnp.where(sel, nhi, ha)
        dc = dc + sel.astype(jnp.int32)
        return h, ha, dc, acc_v, acc_i

    acc_v0 = jnp.zeros((RW, K), jnp.float32)
    acc_i0 = jnp.zeros((RW, K), jnp.int32)
    dc0 = jnp.ones((RW, LANES), jnp.int32)
    _, _, _, acc_v, acc_i = jax.lax.fori_loop(
        0, K // 4, lambda t4, c: extract(
            4 * t4 + 3, extract(4 * t4 + 2, extract(
                4 * t4 + 1, extract(4 * t4, c)))),
        (vs[0], ids[0], dc0, acc_v0, acc_i0))
    rows = pl.ds(r * RW, RW)
    vals_ref[rows, :] = acc_v
    idx_ref[rows, :] = acc_i
    return r + 1


def _body(a_ref, wt_ref, be_ref, dt_ref, bd_ref,
          recon_ref, vals_ref, idx_ref, s_ref, *, ne, mblk, b):
    j = pl.program_id(1)

    @pl.when(j < ne)
    def _encode():
        jj = jnp.minimum(j, ne - 1)
        s = jnp.dot(a_ref[...], wt_ref[...], preferred_element_type=jnp.float32)
        s = s + be_ref[pl.ds(jj * mblk, mblk)][None, :]
        s_ref[:, pl.ds(jj * mblk, mblk)] = jnp.maximum(s, 0.0)

    @pl.when(j == ne)
    def _topk():
        cols = (ne * mblk) // LANES
        jax.lax.fori_loop(
            0, b // RW,
            lambda r, c: _topk_subtile(r, s_ref, vals_ref, idx_ref, cols),
            0)

    @pl.when(j >= ne)
    def _decode():
        jj = jnp.maximum(j - ne, 0)
        v64 = vals_ref[:, pl.ds(K - 1, 1)]
        sb = s_ref[:, pl.ds(jj * mblk, mblk)]
        sm = jnp.where(sb >= v64, sb, 0.0).astype(jnp.bfloat16)
        acc = jnp.dot(sm, dt_ref[...], preferred_element_type=jnp.float32)

        @pl.when(j == ne)
        def _():
            recon_ref[...] = bd_ref[...][None, :] + acc

        @pl.when(j > ne)
        def _():
            recon_ref[...] += acc


def _fused_call(a16, wt, b_e, dt, b_d, *, n, c, m, b, mblk, interpret=False):
    ne = m // mblk
    grid = (n // b, 2 * ne)
    body = functools.partial(_body, ne=ne, mblk=mblk, b=b)
    return pl.pallas_call(
        body,
        grid=grid,
        in_specs=[
            pl.BlockSpec((b, c), lambda i, j: (i, 0)),
            pl.BlockSpec((c, mblk), lambda i, j, ne=ne: (0, jnp.minimum(j, ne - 1))),
            pl.BlockSpec((m,), lambda i, j: (0,)),
            pl.BlockSpec((mblk, c), lambda i, j, ne=ne: (jnp.maximum(j - ne, 0), 0)),
            pl.BlockSpec((c,), lambda i, j: (0,)),
        ],
        out_specs=[
            pl.BlockSpec((b, c), lambda i, j: (i, 0)),
            pl.BlockSpec((b, K), lambda i, j: (i, 0)),
            pl.BlockSpec((b, K), lambda i, j: (i, 0)),
        ],
        out_shape=[
            jax.ShapeDtypeStruct((n, c), jnp.float32),
            jax.ShapeDtypeStruct((n, K), jnp.float32),
            jax.ShapeDtypeStruct((n, K), jnp.int32),
        ],
        scratch_shapes=[pltpu.VMEM((b, m), jnp.float32)],
        compiler_params=pltpu.CompilerParams(
            dimension_semantics=("parallel", "arbitrary")),
        interpret=interpret,
    )(a16, wt, b_e, dt, b_d)


def kernel(a, W_e, b_e, D, b_d, *, interpret=False, b=256, mblk=1024):
    n, c = a.shape
    m = W_e.shape[0]
    a16 = a.astype(jnp.bfloat16)
    wt = W_e.T.astype(jnp.bfloat16)
    dt = D.T.astype(jnp.bfloat16)
    recon, vals, idx = _fused_call(a16, wt, b_e, dt, b_d,
                                   n=n, c=c, m=m, b=b, mblk=mblk,
                                   interpret=interpret)
    return (recon, vals, idx)


# speedup vs baseline: 14.1230x; 1.4032x over previous
import functools

import jax
import jax.numpy as jnp
from jax.experimental import pallas as pl
from jax.experimental.pallas import tpu as pltpu

K = 64
DEPTH = 8
LANES = 128
R = 8


_SORT8 = [(0, 1), (2, 3), (4, 5), (6, 7),
          (0, 2), (1, 3), (4, 6), (5, 7),
          (1, 2), (5, 6),
          (0, 4), (1, 5), (2, 6), (3, 7),
          (2, 4), (3, 5),
          (1, 2), (3, 4), (5, 6)]
_BITONIC8 = [(0, 4), (1, 5), (2, 6), (3, 7),
             (0, 2), (1, 3), (4, 6), (5, 7),
             (0, 1), (2, 3), (4, 5), (6, 7)]


def _cswap(v, iv, i, j):
    ge = v[i] >= v[j]
    vi = jnp.where(ge, v[i], v[j])
    vj = jnp.where(ge, v[j], v[i])
    ii = jnp.where(ge, iv[i], iv[j])
    ij = jnp.where(ge, iv[j], iv[i])
    v[i], v[j], iv[i], iv[j] = vi, vj, ii, ij


RQ = 8
RW = R * RQ


def _insert_lists(rows, s_ref, cols):
    def insert_group(g, carry):
        vs, ids = carry
        slab = s_ref[rows, pl.ds(g * DEPTH * LANES, DEPTH * LANES)]
        xs = [slab[:, t * LANES:(t + 1) * LANES] for t in range(DEPTH)]
        xis = [jnp.full((R, LANES), g * DEPTH + t, jnp.int32)
               for t in range(DEPTH)]
        for i, j in _SORT8:
            _cswap(xs, xis, i, j)
        mv, mi = [], []
        for d in range(DEPTH):
            ge = vs[d] >= xs[DEPTH - 1 - d]
            mv.append(jnp.where(ge, vs[d], xs[DEPTH - 1 - d]))
            mi.append(jnp.where(ge, ids[d], xis[DEPTH - 1 - d]))
        for i, j in _BITONIC8:
            _cswap(mv, mi, i, j)
        return tuple(mv), tuple(mi)

    neg = jnp.full((R, LANES), -1.0, jnp.float32)
    zero_i = jnp.zeros((R, LANES), jnp.int32)
    return jax.lax.fori_loop(
        0, cols // DEPTH, insert_group,
        (tuple(neg for _ in range(DEPTH)), tuple(zero_i for _ in range(DEPTH))))


def _topk_subtile(r, s_ref, vals_ref, idx_ref, cols):
    qlists = [_insert_lists(pl.ds((r * RQ + q) * R, R), s_ref, cols)
              for q in range(RQ)]
    vs = tuple(jnp.concatenate([qlists[q][0][d] for q in range(RQ)], axis=0)
               for d in range(DEPTH))
    ids = tuple(jnp.concatenate([qlists[q][1][d] for q in range(RQ)], axis=0)
                for d in range(DEPTH))

    lane = jax.lax.broadcasted_iota(jnp.int32, (RW, LANES), 1)
    col64 = jax.lax.broadcasted_iota(jnp.int32, (RW, K), 1)
    zero_i = jnp.zeros((RW, LANES), jnp.int32)

    def extract(t, carry):
        h, ha, dc, acc_v, acc_i = carry
        g = jnp.max(h, axis=1, keepdims=True)
        f = ha * LANES + lane
        big = jnp.int32(2 ** 30)
        flat = jnp.min(jnp.where(h == g, f, big), axis=1, keepdims=True)
        sel = (h == g) & (f == flat)
        acc_v = jnp.where(col64 == t, g, acc_v)
        acc_i = jnp.where(col64 == t, flat, acc_i)
        nh = jnp.full((RW, LANES), -1.0, jnp.float32)
        nhi = zero_i
        for d in range(DEPTH - 1, 0, -1):
            is_d = dc == d
            nh = jnp.where(is_d, vs[d], nh)
            nhi = jnp.where(is_d, ids[d], nhi)
        h = jnp.where(sel, nh, h)
        ha = jnp.where(sel, nhi, ha)
        dc = dc + sel.astype(jnp.int32)
        return h, ha, dc, acc_v, acc_i

    acc_v0 = jnp.zeros((RW, K), jnp.float32)
    acc_i0 = jnp.zeros((RW, K), jnp.int32)
    dc0 = jnp.ones((RW, LANES), jnp.int32)
    _, _, _, acc_v, acc_i = jax.lax.fori_loop(
        0, K // 4, lambda t4, c: extract(
            4 * t4 + 3, extract(4 * t4 + 2, extract(
                4 * t4 + 1, extract(4 * t4, c)))),
        (vs[0], ids[0], dc0, acc_v0, acc_i0))
    rows = pl.ds(r * RW, RW)
    vals_ref[rows, :] = acc_v
    idx_ref[rows, :] = acc_i
    return r + 1


def _body(a_ref, wt_ref, be_ref, dt_ref, bd_ref,
          recon_ref, vals_ref, idx_ref, s_ref, *, ne, mblk, b):
    j = pl.program_id(1)

    @pl.when(j < ne)
    def _encode():
        jj = jnp.minimum(j, ne - 1)
        s = jnp.dot(a_ref[...], wt_ref[...], preferred_element_type=jnp.float32)
        s = s + be_ref[pl.ds(jj * mblk, mblk)][None, :]
        s_ref[:, pl.ds(jj * mblk, mblk)] = jnp.maximum(s, 0.0)

    @pl.when(j == ne)
    def _topk():
        cols = (ne * mblk) // LANES
        jax.lax.fori_loop(
            0, b // RW,
            lambda r, c: _topk_subtile(r, s_ref, vals_ref, idx_ref, cols),
            0)

    @pl.when(j >= ne)
    def _decode():
        jj = jnp.maximum(j - ne, 0)
        v64 = vals_ref[:, pl.ds(K - 1, 1)]
        sb = s_ref[:, pl.ds(jj * mblk, mblk)]
        sm = jnp.where(sb >= v64, sb, 0.0).astype(jnp.bfloat16)
        acc = jnp.dot(sm, dt_ref[...], preferred_element_type=jnp.float32)

        @pl.when(j == ne)
        def _():
            recon_ref[...] = bd_ref[...][None, :] + acc

        @pl.when(j > ne)
        def _():
            recon_ref[...] += acc


def _fused_call(a16, wt, b_e, dt, b_d, *, n, c, m, b, mblk, interpret=False):
    ne = m // mblk
    grid = (n // b, 2 * ne)
    body = functools.partial(_body, ne=ne, mblk=mblk, b=b)
    return pl.pallas_call(
        body,
        grid=grid,
        in_specs=[
            pl.BlockSpec((b, c), lambda i, j: (i, 0)),
            pl.BlockSpec((c, mblk), lambda i, j, ne=ne: (0, jnp.minimum(j, ne - 1))),
            pl.BlockSpec((m,), lambda i, j: (0,)),
            pl.BlockSpec((mblk, c), lambda i, j, ne=ne: (jnp.maximum(j - ne, 0), 0)),
            pl.BlockSpec((c,), lambda i, j: (0,)),
        ],
        out_specs=[
            pl.BlockSpec((b, c), lambda i, j: (i, 0)),
            pl.BlockSpec((b, K), lambda i, j: (i, 0)),
            pl.BlockSpec((b, K), lambda i, j: (i, 0)),
        ],
        out_shape=[
            jax.ShapeDtypeStruct((n, c), jnp.float32),
            jax.ShapeDtypeStruct((n, K), jnp.float32),
            jax.ShapeDtypeStruct((n, K), jnp.int32),
        ],
        scratch_shapes=[pltpu.VMEM((b, m), jnp.float32)],
        compiler_params=pltpu.CompilerParams(
            dimension_semantics=("parallel", "arbitrary")),
        interpret=interpret,
    )(a16, wt, b_e, dt, b_d)


def kernel(a, W_e, b_e, D, b_d, *, interpret=False, b=256, mblk=1024):
    n, c = a.shape
    m = W_e.shape[0]
    a16 = a.astype(jnp.bfloat16)
    wt = W_e.T.astype(jnp.bfloat16)
    dt = D.T.astype(jnp.bfloat16)
    recon, vals, idx = _fused_call(a16, wt, b_e, dt, b_d,
                                   n=n, c=c, m=m, b=b, mblk=mblk,
                                   interpret=interpret)
    return (recon, vals, idx)


# speedup vs baseline: 17.6552x; 1.2501x over previous
import functools

import jax
import jax.numpy as jnp
from jax.experimental import pallas as pl
from jax.experimental.pallas import tpu as pltpu

K = 64
DEPTH = 8
LANES = 128
R = 8


_SORT8 = [(0, 1), (2, 3), (4, 5), (6, 7),
          (0, 2), (1, 3), (4, 6), (5, 7),
          (1, 2), (5, 6),
          (0, 4), (1, 5), (2, 6), (3, 7),
          (2, 4), (3, 5),
          (1, 2), (3, 4), (5, 6)]
_BITONIC8 = [(0, 4), (1, 5), (2, 6), (3, 7),
             (0, 2), (1, 3), (4, 6), (5, 7),
             (0, 1), (2, 3), (4, 5), (6, 7)]


def _cswap(v, iv, i, j):
    ge = v[i] >= v[j]
    vi = jnp.where(ge, v[i], v[j])
    vj = jnp.where(ge, v[j], v[i])
    ii = jnp.where(ge, iv[i], iv[j])
    ij = jnp.where(ge, iv[j], iv[i])
    v[i], v[j], iv[i], iv[j] = vi, vj, ii, ij


RQ = 16
RW = R * RQ


def _insert_lists(rows, s_ref, cols):
    def insert_group(g, carry):
        vs, ids = carry
        slab = s_ref[rows, pl.ds(g * DEPTH * LANES, DEPTH * LANES)]
        xs = [slab[:, t * LANES:(t + 1) * LANES] for t in range(DEPTH)]
        xis = [jnp.full((R, LANES), g * DEPTH + t, jnp.int32)
               for t in range(DEPTH)]
        for i, j in _SORT8:
            _cswap(xs, xis, i, j)
        mv, mi = [], []
        for d in range(DEPTH):
            ge = vs[d] >= xs[DEPTH - 1 - d]
            mv.append(jnp.where(ge, vs[d], xs[DEPTH - 1 - d]))
            mi.append(jnp.where(ge, ids[d], xis[DEPTH - 1 - d]))
        for i, j in _BITONIC8:
            _cswap(mv, mi, i, j)
        return tuple(mv), tuple(mi)

    neg = jnp.full((R, LANES), -1.0, jnp.float32)
    zero_i = jnp.zeros((R, LANES), jnp.int32)
    return jax.lax.fori_loop(
        0, cols // DEPTH, insert_group,
        (tuple(neg for _ in range(DEPTH)), tuple(zero_i for _ in range(DEPTH))))


def _topk_subtile(r, s_ref, vals_ref, idx_ref, cols):
    qlists = [_insert_lists(pl.ds((r * RQ + q) * R, R), s_ref, cols)
              for q in range(RQ)]
    vs = tuple(jnp.concatenate([qlists[q][0][d] for q in range(RQ)], axis=0)
               for d in range(DEPTH))
    ids = tuple(jnp.concatenate([qlists[q][1][d] for q in range(RQ)], axis=0)
                for d in range(DEPTH))

    lane = jax.lax.broadcasted_iota(jnp.int32, (RW, LANES), 1)
    col64 = jax.lax.broadcasted_iota(jnp.int32, (RW, K), 1)
    zero_i = jnp.zeros((RW, LANES), jnp.int32)

    def extract(t, carry):
        h, ha, dc, acc_v, acc_i = carry
        g = jnp.max(h, axis=1, keepdims=True)
        f = ha * LANES + lane
        big = jnp.int32(2 ** 30)
        flat = jnp.min(jnp.where(h == g, f, big), axis=1, keepdims=True)
        sel = (h == g) & (f == flat)
        acc_v = jnp.where(col64 == t, g, acc_v)
        acc_i = jnp.where(col64 == t, flat, acc_i)
        nh = jnp.full((RW, LANES), -1.0, jnp.float32)
        nhi = zero_i
        for d in range(DEPTH - 1, 0, -1):
            is_d = dc == d
            nh = jnp.where(is_d, vs[d], nh)
            nhi = jnp.where(is_d, ids[d], nhi)
        h = jnp.where(sel, nh, h)
        ha = jnp.where(sel, nhi, ha)
        dc = dc + sel.astype(jnp.int32)
        return h, ha, dc, acc_v, acc_i

    acc_v0 = jnp.zeros((RW, K), jnp.float32)
    acc_i0 = jnp.zeros((RW, K), jnp.int32)
    dc0 = jnp.ones((RW, LANES), jnp.int32)
    _, _, _, acc_v, acc_i = jax.lax.fori_loop(
        0, K // 4, lambda t4, c: extract(
            4 * t4 + 3, extract(4 * t4 + 2, extract(
                4 * t4 + 1, extract(4 * t4, c)))),
        (vs[0], ids[0], dc0, acc_v0, acc_i0))
    rows = pl.ds(r * RW, RW)
    vals_ref[rows, :] = acc_v
    idx_ref[rows, :] = acc_i
    return r + 1


def _body(a_ref, wt_ref, be_ref, dt_ref, bd_ref,
          recon_ref, vals_ref, idx_ref, s_ref, *, ne, mblk, b):
    j = pl.program_id(1)

    @pl.when(j < ne)
    def _encode():
        jj = jnp.minimum(j, ne - 1)
        s = jnp.dot(a_ref[...], wt_ref[...], preferred_element_type=jnp.float32)
        s = s + be_ref[pl.ds(jj * mblk, mblk)][None, :]
        s_ref[:, pl.ds(jj * mblk, mblk)] = jnp.maximum(s, 0.0)

    @pl.when(j == ne)
    def _topk():
        cols = (ne * mblk) // LANES
        jax.lax.fori_loop(
            0, b // RW,
            lambda r, c: _topk_subtile(r, s_ref, vals_ref, idx_ref, cols),
            0)

    @pl.when(j >= ne)
    def _decode():
        jj = jnp.maximum(j - ne, 0)
        v64 = vals_ref[:, pl.ds(K - 1, 1)]
        sb = s_ref[:, pl.ds(jj * mblk, mblk)]
        sm = jnp.where(sb >= v64, sb, 0.0).astype(jnp.bfloat16)
        acc = jnp.dot(sm, dt_ref[...], preferred_element_type=jnp.float32)

        @pl.when(j == ne)
        def _():
            recon_ref[...] = bd_ref[...][None, :] + acc

        @pl.when(j > ne)
        def _():
            recon_ref[...] += acc


def _fused_call(a16, wt, b_e, dt, b_d, *, n, c, m, b, mblk, interpret=False):
    ne = m // mblk
    grid = (n // b, 2 * ne)
    body = functools.partial(_body, ne=ne, mblk=mblk, b=b)
    return pl.pallas_call(
        body,
        grid=grid,
        in_specs=[
            pl.BlockSpec((b, c), lambda i, j: (i, 0)),
            pl.BlockSpec((c, mblk), lambda i, j, ne=ne: (0, jnp.minimum(j, ne - 1))),
            pl.BlockSpec((m,), lambda i, j: (0,)),
            pl.BlockSpec((mblk, c), lambda i, j, ne=ne: (jnp.maximum(j - ne, 0), 0)),
            pl.BlockSpec((c,), lambda i, j: (0,)),
        ],
        out_specs=[
            pl.BlockSpec((b, c), lambda i, j: (i, 0)),
            pl.BlockSpec((b, K), lambda i, j: (i, 0)),
            pl.BlockSpec((b, K), lambda i, j: (i, 0)),
        ],
        out_shape=[
            jax.ShapeDtypeStruct((n, c), jnp.float32),
            jax.ShapeDtypeStruct((n, K), jnp.float32),
            jax.ShapeDtypeStruct((n, K), jnp.int32),
        ],
        scratch_shapes=[pltpu.VMEM((b, m), jnp.float32)],
        compiler_params=pltpu.CompilerParams(
            dimension_semantics=("parallel", "arbitrary")),
        interpret=interpret,
    )(a16, wt, b_e, dt, b_d)


def kernel(a, W_e, b_e, D, b_d, *, interpret=False, b=256, mblk=1024):
    n, c = a.shape
    m = W_e.shape[0]
    a16 = a.astype(jnp.bfloat16)
    wt = W_e.T.astype(jnp.bfloat16)
    dt = D.T.astype(jnp.bfloat16)
    recon, vals, idx = _fused_call(a16, wt, b_e, dt, b_d,
                                   n=n, c=c, m=m, b=b, mblk=mblk,
                                   interpret=interpret)
    return (recon, vals, idx)
